# baseline (speedup 1.0000x reference)
import jax
import jax.numpy as jnp
from jax import lax
from jax.experimental import pallas as pl
from jax.experimental.pallas import tpu as pltpu
from jax.experimental.pallas import tpu_sc as plsc

_SHAPE = (64, 8192)
_TABLE = _SHAPE[1]

_info = plsc.get_sparse_core_info()
_NC, _NS, _L = _info.num_cores, _info.num_subcores, _info.num_lanes
_NW = _NC * _NS
_BR, _BC = 8, 2048
_RG = _SHAPE[0] // _BR
_CG = _SHAPE[1] // _BC
_NSUB = 4
_SCC = _BC // _NSUB
_SUBV = _SCC // _L


def _gather_body(x_hbm, idx_hbm, out_hbm, table_v, idx_v, out_v,
                 sem_t, sem_i, sem_o):
    wid = lax.axis_index("s") * _NC + lax.axis_index("c")
    r0 = (wid // _CG) * _BR
    c0 = (wid % _CG) * _BC
    table_cp = pltpu.async_copy(x_hbm, table_v, sem_t)
    idx_cp = pltpu.async_copy(
        idx_hbm.at[pl.ds(r0, _BR), pl.ds(c0, _SCC)],
        idx_v.at[:, pl.ds(0, _SCC)], sem_i)
    table_cp.wait()
    out_cps = []
    for k in range(_NSUB):
        idx_cp.wait()
        if k + 1 < _NSUB:
            cn = (k + 1) * _SCC
            idx_cp = pltpu.async_copy(
                idx_hbm.at[pl.ds(r0, _BR), pl.ds(c0 + cn, _SCC)],
                idx_v.at[:, pl.ds(cn, _SCC)], sem_i)
        ck = k * _SCC

        @plsc.parallel_loop(0, _BR * _SUBV, unroll=16)
        def step(i, _ck=ck):
            r = i // _SUBV
            off = _ck + (i % _SUBV) * _L
            iv = idx_v[r, pl.ds(off, _L)]
            out_v[r, pl.ds(off, _L)] = plsc.load_gather(table_v, [iv])

        out_cps.append(pltpu.async_copy(
            out_v.at[:, pl.ds(ck, _SCC)],
            out_hbm.at[pl.ds(r0, _BR), pl.ds(c0 + ck, _SCC)], sem_o))
    for cp in out_cps:
        cp.wait()


_gather = pl.kernel(
    _gather_body,
    out_type=jax.ShapeDtypeStruct(_SHAPE, jnp.float32),
    mesh=plsc.VectorSubcoreMesh(core_axis_name="c", subcore_axis_name="s"),
    scratch_types=[
        pltpu.VMEM((_TABLE,), jnp.float32),
        pltpu.VMEM((_BR, _BC), jnp.int32),
        pltpu.VMEM((_BR, _BC), jnp.float32),
        pltpu.SemaphoreType.DMA,
        pltpu.SemaphoreType.DMA,
        pltpu.SemaphoreType.DMA,
    ],
    compiler_params=pltpu.CompilerParams(
        needs_layout_passes=False, use_tc_tiling_on_sc=True),
)


def kernel(x, idx_tensor):
    return _gather(x, idx_tensor.astype(jnp.int32))


# speedup vs baseline: 1.0622x; 1.0622x over previous
import jax
import jax.numpy as jnp
from jax import lax
from jax.experimental import pallas as pl
from jax.experimental.pallas import tpu as pltpu
from jax.experimental.pallas import tpu_sc as plsc

_SHAPE = (64, 8192)
_TABLE = _SHAPE[1]

_info = plsc.get_sparse_core_info()
_NC, _NS, _L = _info.num_cores, _info.num_subcores, _info.num_lanes
_NW = _NC * _NS
_BR, _BC = 8, 2048
_RG = _SHAPE[0] // _BR
_CG = _SHAPE[1] // _BC
_NSUB = 2
_SCC = _BC // _NSUB
_SUBV = _SCC // _L


def _gather_body(x_hbm, idx_hbm, out_hbm, table_v, idx_v, out_v,
                 sem_t, sem_i, sem_o):
    wid = lax.axis_index("s") * _NC + lax.axis_index("c")
    r0 = (wid // _CG) * _BR
    c0 = (wid % _CG) * _BC
    table_cp = pltpu.async_copy(x_hbm, table_v, sem_t)
    idx_cp = pltpu.async_copy(
        idx_hbm.at[pl.ds(r0, _BR), pl.ds(c0, _SCC)],
        idx_v.at[:, pl.ds(0, _SCC)], sem_i)
    table_cp.wait()
    out_cps = []
    for k in range(_NSUB):
        idx_cp.wait()
        if k + 1 < _NSUB:
            cn = (k + 1) * _SCC
            idx_cp = pltpu.async_copy(
                idx_hbm.at[pl.ds(r0, _BR), pl.ds(c0 + cn, _SCC)],
                idx_v.at[:, pl.ds(cn, _SCC)], sem_i)
        ck = k * _SCC

        @plsc.parallel_loop(0, _BR * _SUBV, unroll=8)
        def step(i, _ck=ck):
            r = i // _SUBV
            off = _ck + (i % _SUBV) * _L
            iv = idx_v[r, pl.ds(off, _L)]
            out_v[r, pl.ds(off, _L)] = plsc.load_gather(table_v, [iv])

        out_cps.append(pltpu.async_copy(
            out_v.at[:, pl.ds(ck, _SCC)],
            out_hbm.at[pl.ds(r0, _BR), pl.ds(c0 + ck, _SCC)], sem_o))
    for cp in out_cps:
        cp.wait()


_gather = pl.kernel(
    _gather_body,
    out_type=jax.ShapeDtypeStruct(_SHAPE, jnp.float32),
    mesh=plsc.VectorSubcoreMesh(core_axis_name="c", subcore_axis_name="s"),
    scratch_types=[
        pltpu.VMEM((_TABLE,), jnp.float32),
        pltpu.VMEM((_BR, _BC), jnp.int32),
        pltpu.VMEM((_BR, _BC), jnp.float32),
        pltpu.SemaphoreType.DMA,
        pltpu.SemaphoreType.DMA,
        pltpu.SemaphoreType.DMA,
    ],
    compiler_params=pltpu.CompilerParams(
        needs_layout_passes=False, use_tc_tiling_on_sc=True),
)


def kernel(x, idx_tensor):
    return _gather(x, idx_tensor.astype(jnp.int32))


# speedup vs baseline: 1.0913x; 1.0273x over previous
import jax
import jax.numpy as jnp
from jax import lax
from jax.experimental import pallas as pl
from jax.experimental.pallas import tpu as pltpu
from jax.experimental.pallas import tpu_sc as plsc

_SHAPE = (64, 8192)
_TABLE = _SHAPE[1]

_info = plsc.get_sparse_core_info()
_NC, _NS, _L = _info.num_cores, _info.num_subcores, _info.num_lanes
_NW = _NC * _NS
_BR, _BC = 8, 2048
_RG = _SHAPE[0] // _BR
_CG = _SHAPE[1] // _BC
_NSUB = 1
_SCC = _BC // _NSUB
_SUBV = _SCC // _L


def _gather_body(x_hbm, idx_hbm, out_hbm, table_v, idx_v, out_v,
                 sem_t, sem_i, sem_o):
    wid = lax.axis_index("s") * _NC + lax.axis_index("c")
    r0 = (wid // _CG) * _BR
    c0 = (wid % _CG) * _BC
    table_cp = pltpu.async_copy(x_hbm, table_v, sem_t)
    idx_cp = pltpu.async_copy(
        idx_hbm.at[pl.ds(r0, _BR), pl.ds(c0, _SCC)],
        idx_v.at[:, pl.ds(0, _SCC)], sem_i)
    table_cp.wait()
    out_cps = []
    for k in range(_NSUB):
        idx_cp.wait()
        if k + 1 < _NSUB:
            cn = (k + 1) * _SCC
            idx_cp = pltpu.async_copy(
                idx_hbm.at[pl.ds(r0, _BR), pl.ds(c0 + cn, _SCC)],
                idx_v.at[:, pl.ds(cn, _SCC)], sem_i)
        ck = k * _SCC

        @plsc.parallel_loop(0, _BR * _SUBV, unroll=8)
        def step(i, _ck=ck):
            r = i // _SUBV
            off = _ck + (i % _SUBV) * _L
            iv = idx_v[r, pl.ds(off, _L)]
            out_v[r, pl.ds(off, _L)] = plsc.load_gather(table_v, [iv])

        out_cps.append(pltpu.async_copy(
            out_v.at[:, pl.ds(ck, _SCC)],
            out_hbm.at[pl.ds(r0, _BR), pl.ds(c0 + ck, _SCC)], sem_o))
    for cp in out_cps:
        cp.wait()


_gather = pl.kernel(
    _gather_body,
    out_type=jax.ShapeDtypeStruct(_SHAPE, jnp.float32),
    mesh=plsc.VectorSubcoreMesh(core_axis_name="c", subcore_axis_name="s"),
    scratch_types=[
        pltpu.VMEM((_TABLE,), jnp.float32),
        pltpu.VMEM((_BR, _BC), jnp.int32),
        pltpu.VMEM((_BR, _BC), jnp.float32),
        pltpu.SemaphoreType.DMA,
        pltpu.SemaphoreType.DMA,
        pltpu.SemaphoreType.DMA,
    ],
    compiler_params=pltpu.CompilerParams(
        needs_layout_passes=False, use_tc_tiling_on_sc=True),
)


def kernel(x, idx_tensor):
    return _gather(x, idx_tensor.astype(jnp.int32))


# speedup vs baseline: 1.0925x; 1.0012x over previous
import jax
import jax.numpy as jnp
from jax import lax
from jax.experimental import pallas as pl
from jax.experimental.pallas import tpu as pltpu
from jax.experimental.pallas import tpu_sc as plsc

_SHAPE = (64, 8192)
_TABLE = _SHAPE[1]

_info = plsc.get_sparse_core_info()
_NC, _NS, _L = _info.num_cores, _info.num_subcores, _info.num_lanes
_NW = _NC * _NS
_BR, _BC = 8, 2048
_CG = _SHAPE[1] // _BC
_RVECS = _BC // _L


def _gather_body(x_hbm, idx_hbm, out_hbm, table_v, idx_v, out_v, sem):
    wid = lax.axis_index("s") * _NC + lax.axis_index("c")
    r0 = (wid // _CG) * _BR
    c0 = (wid % _CG) * _BC
    table_cp = pltpu.async_copy(x_hbm, table_v, sem)
    idx_cp = pltpu.async_copy(
        idx_hbm.at[pl.ds(r0, _BR), pl.ds(c0, _BC)], idx_v, sem)
    table_cp.wait()
    idx_cp.wait()

    @plsc.parallel_loop(0, _BR * _RVECS, unroll=8)
    def step(i):
        r = i // _RVECS
        off = (i % _RVECS) * _L
        iv = idx_v[r, pl.ds(off, _L)]
        out_v[r, pl.ds(off, _L)] = plsc.load_gather(table_v, [iv])

    pltpu.async_copy(
        out_v, out_hbm.at[pl.ds(r0, _BR), pl.ds(c0, _BC)], sem).wait()


_gather = pl.kernel(
    _gather_body,
    out_type=jax.ShapeDtypeStruct(_SHAPE, jnp.float32),
    mesh=plsc.VectorSubcoreMesh(core_axis_name="c", subcore_axis_name="s"),
    scratch_types=[
        pltpu.VMEM((_TABLE,), jnp.float32),
        pltpu.VMEM((_BR, _BC), jnp.int32),
        pltpu.VMEM((_BR, _BC), jnp.float32),
        pltpu.SemaphoreType.DMA,
    ],
    compiler_params=pltpu.CompilerParams(
        needs_layout_passes=False, use_tc_tiling_on_sc=True),
)


def kernel(x, idx_tensor):
    return _gather(x, idx_tensor.astype(jnp.int32))
